# Initial kernel scaffold; baseline (speedup 1.0000x reference)
import jax
import jax.numpy as jnp
from jax.experimental import pallas as pl


def kernel(x, edge_index, edge_weight, W1, b1, W2, b2, W3, b3):
    raise NotImplementedError("write your pallas kernel here")


# speedup vs baseline: 7.8135x; 7.8135x over previous
import functools

import jax
import jax.numpy as jnp
from jax import lax
from jax.experimental import pallas as pl
from jax.experimental.pallas import tpu as pltpu
from jax.experimental.pallas import tpu_sc as plsc

N = 10000
E = 320000
D_IN = 128
H = 64
D_OUT = 128

NC = 2
NS = 16
NW = NC * NS
CHUNK = 128
CHUNKS_PER_W = (E + NW * CHUNK - 1) // (NW * CHUNK)
E_PAD = NW * CHUNK * CHUNKS_PER_W
EPW = CHUNK * CHUNKS_PER_W
N_PAD = 10240
ROWS_PER_S = N_PAD // NS

_mesh = plsc.VectorSubcoreMesh(core_axis_name="c", subcore_axis_name="s")


@functools.partial(
    pl.kernel,
    out_type=jax.ShapeDtypeStruct((NC, N_PAD), jnp.float32),
    mesh=_mesh,
    scratch_types=[
        pltpu.VMEM_SHARED((N_PAD,), jnp.float32),
        pltpu.VMEM((CHUNK,), jnp.int32),
        pltpu.VMEM((CHUNK,), jnp.float32),
        pltpu.VMEM((ROWS_PER_S,), jnp.float32),
    ],
)
def _sc_degree(dst_hbm, w_hbm, out_hbm, acc, dstbuf, wbuf, zbuf):
    c = lax.axis_index("c")
    s = lax.axis_index("s")
    wid = s * NC + c

    def _z(i, _):
        zbuf[pl.ds(i * 16, 16)] = jnp.zeros((16,), jnp.float32)
        return 0
    lax.fori_loop(0, ROWS_PER_S // 16, _z, 0)
    pltpu.sync_copy(zbuf, acc.at[pl.ds(s * ROWS_PER_S, ROWS_PER_S)])
    plsc.subcore_barrier()

    base = wid * EPW

    def _chunk(ci, _):
        off = base + ci * CHUNK
        pltpu.sync_copy(dst_hbm.at[pl.ds(off, CHUNK)], dstbuf)
        pltpu.sync_copy(w_hbm.at[pl.ds(off, CHUNK)], wbuf)
        pltpu.sync_copy(wbuf, acc.at[dstbuf], add=True)
        return 0
    lax.fori_loop(0, CHUNKS_PER_W, _chunk, 0)

    plsc.subcore_barrier()
    pltpu.sync_copy(acc.at[pl.ds(s * ROWS_PER_S, ROWS_PER_S)],
                    out_hbm.at[c, pl.ds(s * ROWS_PER_S, ROWS_PER_S)])


@functools.partial(
    pl.kernel,
    out_type=jax.ShapeDtypeStruct((NC, N_PAD, H), jnp.float32),
    mesh=_mesh,
    scratch_types=[
        pltpu.VMEM_SHARED((N_PAD, H), jnp.float32),
        pltpu.VMEM((CHUNK,), jnp.int32),
        pltpu.VMEM((CHUNK,), jnp.int32),
        pltpu.VMEM((CHUNK,), jnp.float32),
        pltpu.VMEM((CHUNK, H), jnp.float32),
        pltpu.SemaphoreType.DMA,
    ],
    compiler_params=pltpu.CompilerParams(use_tc_tiling_on_sc=False),
)
def _sc_aggregate(g_hbm, src_hbm, dst_hbm, w_hbm, out_hbm,
                  acc, srcbuf, dstbuf, wbuf, rows, sem):
    c = lax.axis_index("c")
    s = lax.axis_index("s")
    wid = s * NC + c

    def _z(i, _):
        for j in range(H // 16):
            rows[i, pl.ds(j * 16, 16)] = jnp.zeros((16,), jnp.float32)
        return 0
    lax.fori_loop(0, CHUNK, _z, 0)
    for k in range(ROWS_PER_S // CHUNK):
        pltpu.sync_copy(rows, acc.at[pl.ds(s * ROWS_PER_S + k * CHUNK, CHUNK)])
    plsc.subcore_barrier()

    base = wid * EPW

    def _chunk(ci, _):
        off = base + ci * CHUNK
        pltpu.sync_copy(src_hbm.at[pl.ds(off, CHUNK)], srcbuf)
        pltpu.async_copy(g_hbm.at[srcbuf], rows, sem).wait()
        pltpu.sync_copy(dst_hbm.at[pl.ds(off, CHUNK)], dstbuf)
        pltpu.sync_copy(w_hbm.at[pl.ds(off, CHUNK)], wbuf)

        def _scale(gi, _):
            wv = wbuf[pl.ds(gi * 16, 16)]
            for el in range(16):
                e = gi * 16 + el
                we = wv[el]
                for j in range(H // 16):
                    sl = pl.ds(j * 16, 16)
                    rows[e, sl] = rows[e, sl] * we
            return 0
        lax.fori_loop(0, CHUNK // 16, _scale, 0)

        pltpu.sync_copy(rows, acc.at[dstbuf], add=True)
        return 0
    lax.fori_loop(0, CHUNKS_PER_W, _chunk, 0)

    plsc.subcore_barrier()
    pltpu.sync_copy(acc.at[pl.ds(s * ROWS_PER_S, ROWS_PER_S)],
                    out_hbm.at[c, pl.ds(s * ROWS_PER_S, ROWS_PER_S)])


R = 2000
G = N // R


def _tc_first(deg_ref, x_ref, w1_ref, t1_ref, g1_ref, dinv_ref, dinv2_ref):
    deg = deg_ref[0] + deg_ref[1] + 1.0
    dinv = jnp.where(deg > 0,
                     lax.rsqrt(jnp.maximum(deg, 1e-12)), 0.0)
    t = jnp.dot(x_ref[...], w1_ref[...],
                preferred_element_type=jnp.float32)
    t1_ref[...] = t
    g1_ref[...] = t * dinv
    dinv_ref[...] = dinv
    dinv2_ref[...] = dinv * dinv


def _tc_mid(accp_ref, t_ref, dinv_ref, dinv2_ref, b_ref, w_ref,
            tn_ref, gn_ref):
    a = (dinv_ref[...] * (accp_ref[0] + accp_ref[1])
         + dinv2_ref[...] * t_ref[...] + b_ref[...])
    h = jnp.maximum(a, 0.0)
    t = jnp.dot(h, w_ref[...], preferred_element_type=jnp.float32)
    tn_ref[...] = t
    gn_ref[...] = t * dinv_ref[...]


def _tc_pre_final(accp_ref, t_ref, dinv_ref, dinv2_ref, b_ref,
                  h_ref, gn_ref):
    a = (dinv_ref[...] * (accp_ref[0] + accp_ref[1])
         + dinv2_ref[...] * t_ref[...] + b_ref[...])
    h = jnp.maximum(a, 0.0)
    h_ref[...] = h
    gn_ref[...] = h * dinv_ref[...]


def _tc_final(accp_ref, h_ref, dinv_ref, dinv2_ref, b_ref, w_ref, out_ref):
    a = (dinv_ref[...] * (accp_ref[0] + accp_ref[1])
         + dinv2_ref[...] * h_ref[...])
    out_ref[...] = (jnp.dot(a, w_ref[...], preferred_element_type=jnp.float32)
                    + b_ref[...])


def _rows(block_shape):
    return pl.BlockSpec(block_shape, lambda i: (i,) + (0,) * (len(block_shape) - 1))


def _accp_spec():
    return pl.BlockSpec((NC, R, H), lambda i: (0, i, 0))


def _full(shape):
    return pl.BlockSpec(shape, lambda i: (0,) * len(shape))


def kernel(x, edge_index, edge_weight, W1, b1, W2, b2, W3, b3):
    src = edge_index[0]
    dst = edge_index[1]
    pad = E_PAD - E
    src = jnp.concatenate([src, jnp.zeros((pad,), jnp.int32)])
    dst = jnp.concatenate([dst, jnp.zeros((pad,), jnp.int32)])
    w = jnp.concatenate([edge_weight, jnp.zeros((pad,), jnp.float32)])

    degp = _sc_degree(dst, w)
    degp3 = degp.reshape(NC, N_PAD, 1)

    b1r = b1.reshape(1, H)
    b2r = b2.reshape(1, H)
    b3r = b3.reshape(1, D_OUT)

    t1, g1, dinv, dinv2 = pl.pallas_call(
        _tc_first,
        grid=(G,),
        in_specs=[pl.BlockSpec((NC, R, 1), lambda i: (0, i, 0)),
                  _rows((R, D_IN)), _full((D_IN, H))],
        out_specs=[_rows((R, H)), _rows((R, H)),
                   _rows((R, 1)), _rows((R, 1))],
        out_shape=[jax.ShapeDtypeStruct((N, H), jnp.float32),
                   jax.ShapeDtypeStruct((N, H), jnp.float32),
                   jax.ShapeDtypeStruct((N, 1), jnp.float32),
                   jax.ShapeDtypeStruct((N, 1), jnp.float32)],
    )(degp3, x, W1)

    acc1 = _sc_aggregate(g1, src, dst, w)

    t2, g2 = pl.pallas_call(
        _tc_mid,
        grid=(G,),
        in_specs=[_accp_spec(), _rows((R, H)), _rows((R, 1)),
                  _rows((R, 1)), _full((1, H)), _full((H, H))],
        out_specs=[_rows((R, H)), _rows((R, H))],
        out_shape=[jax.ShapeDtypeStruct((N, H), jnp.float32),
                   jax.ShapeDtypeStruct((N, H), jnp.float32)],
    )(acc1, t1, dinv, dinv2, b1r, W2)

    acc2 = _sc_aggregate(g2, src, dst, w)

    h2, g3 = pl.pallas_call(
        _tc_pre_final,
        grid=(G,),
        in_specs=[_accp_spec(), _rows((R, H)), _rows((R, 1)),
                  _rows((R, 1)), _full((1, H))],
        out_specs=[_rows((R, H)), _rows((R, H))],
        out_shape=[jax.ShapeDtypeStruct((N, H), jnp.float32),
                   jax.ShapeDtypeStruct((N, H), jnp.float32)],
    )(acc2, t2, dinv, dinv2, b2r)

    acc3 = _sc_aggregate(g3, src, dst, w)

    out = pl.pallas_call(
        _tc_final,
        grid=(G,),
        in_specs=[_accp_spec(), _rows((R, H)), _rows((R, 1)),
                  _rows((R, 1)), _full((1, D_OUT)), _full((H, D_OUT))],
        out_specs=_rows((R, D_OUT)),
        out_shape=jax.ShapeDtypeStruct((N, D_OUT), jnp.float32),
    )(acc3, h2, dinv, dinv2, b3r, W3)

    return out


# speedup vs baseline: 12.4749x; 1.5966x over previous
import functools

import jax
import jax.numpy as jnp
from jax import lax
from jax.experimental import pallas as pl
from jax.experimental.pallas import tpu as pltpu
from jax.experimental.pallas import tpu_sc as plsc

N = 10000
E = 320000
D_IN = 128
H = 64
D_OUT = 128

NC = 2
NS = 16
NW = NC * NS
CHUNK = 128
CPW = 80
EPW = CHUNK * CPW
E_PAD = NW * EPW
N_PAD = 10240
ROWS_PER_S = N_PAD // NS

_mesh = plsc.VectorSubcoreMesh(core_axis_name="c", subcore_axis_name="s")


@functools.partial(
    pl.kernel,
    out_type=jax.ShapeDtypeStruct((NC, N_PAD), jnp.float32),
    mesh=_mesh,
    scratch_types=[
        pltpu.VMEM_SHARED((N_PAD,), jnp.float32),
        pltpu.VMEM((CPW, CHUNK), jnp.int32),
        pltpu.VMEM((CPW, CHUNK), jnp.float32),
        pltpu.VMEM((ROWS_PER_S,), jnp.float32),
    ],
    compiler_params=pltpu.CompilerParams(use_tc_tiling_on_sc=False),
)
def _sc_degree(dst_hbm, w_hbm, out_hbm, acc, dstb, wb, zbuf):
    c = lax.axis_index("c")
    s = lax.axis_index("s")
    wid = s * NC + c

    def _z(i, _):
        zbuf[pl.ds(i * 16, 16)] = jnp.zeros((16,), jnp.float32)
        return 0
    lax.fori_loop(0, ROWS_PER_S // 16, _z, 0)
    pltpu.sync_copy(zbuf, acc.at[pl.ds(s * ROWS_PER_S, ROWS_PER_S)])

    pltpu.sync_copy(dst_hbm.at[pl.ds(wid * CPW, CPW)], dstb)
    pltpu.sync_copy(w_hbm.at[pl.ds(wid * CPW, CPW)], wb)
    plsc.subcore_barrier()

    def _chunk(ci, _):
        pltpu.sync_copy(wb.at[ci], acc.at[dstb.at[ci]], add=True)
        return 0
    lax.fori_loop(0, CPW, _chunk, 0)

    plsc.subcore_barrier()
    pltpu.sync_copy(acc.at[pl.ds(s * ROWS_PER_S, ROWS_PER_S)],
                    out_hbm.at[c, pl.ds(s * ROWS_PER_S, ROWS_PER_S)])


@functools.partial(
    pl.kernel,
    out_type=jax.ShapeDtypeStruct((NC, N_PAD, H), jnp.float32),
    mesh=_mesh,
    scratch_types=[
        pltpu.VMEM_SHARED((N_PAD, H), jnp.float32),
        pltpu.VMEM((CPW, CHUNK), jnp.int32),
        pltpu.VMEM((CPW, CHUNK), jnp.int32),
        pltpu.VMEM((CPW, CHUNK), jnp.float32),
        pltpu.VMEM((CHUNK, H), jnp.float32),
        pltpu.VMEM((CHUNK, H), jnp.float32),
        pltpu.SemaphoreType.DMA,
        pltpu.SemaphoreType.DMA,
    ],
    compiler_params=pltpu.CompilerParams(use_tc_tiling_on_sc=False),
)
def _sc_aggregate(g_hbm, src_hbm, dst_hbm, w_hbm, out_hbm,
                  acc, srcb, dstb, wb, rows0, rows1, sem0, sem1):
    c = lax.axis_index("c")
    s = lax.axis_index("s")
    wid = s * NC + c

    def _z(i, _):
        for j in range(H // 16):
            rows0[i, pl.ds(j * 16, 16)] = jnp.zeros((16,), jnp.float32)
        return 0
    lax.fori_loop(0, CHUNK, _z, 0)
    for k in range(ROWS_PER_S // CHUNK):
        pltpu.sync_copy(rows0, acc.at[pl.ds(s * ROWS_PER_S + k * CHUNK, CHUNK)])

    pltpu.sync_copy(src_hbm.at[pl.ds(wid * CPW, CPW)], srcb)
    pltpu.sync_copy(dst_hbm.at[pl.ds(wid * CPW, CPW)], dstb)
    pltpu.sync_copy(w_hbm.at[pl.ds(wid * CPW, CPW)], wb)
    plsc.subcore_barrier()

    pltpu.async_copy(g_hbm.at[srcb.at[0]], rows0, sem0)
    pltpu.async_copy(g_hbm.at[srcb.at[1]], rows1, sem1)

    def _pair(i, _):
        for b, (rows, sem) in enumerate(((rows0, sem0), (rows1, sem1))):
            ci = 2 * i + b
            pltpu.make_async_copy(g_hbm.at[srcb.at[ci]], rows, sem).wait()

            def _scale(gi, _):
                wv = wb[ci, pl.ds(gi * 16, 16)]
                for el in range(16):
                    e = gi * 16 + el
                    we = wv[el]
                    for j in range(H // 16):
                        sl = pl.ds(j * 16, 16)
                        rows[e, sl] = rows[e, sl] * we
                return 0
            lax.fori_loop(0, CHUNK // 16, _scale, 0)

            pltpu.sync_copy(rows, acc.at[dstb.at[ci]], add=True)

            @pl.when(ci + 2 < CPW)
            def _():
                pltpu.async_copy(g_hbm.at[srcb.at[ci + 2]], rows, sem)
        return 0
    lax.fori_loop(0, CPW // 2, _pair, 0)

    plsc.subcore_barrier()
    pltpu.sync_copy(acc.at[pl.ds(s * ROWS_PER_S, ROWS_PER_S)],
                    out_hbm.at[c, pl.ds(s * ROWS_PER_S, ROWS_PER_S)])


R = 2000
G = N // R


def _tc_first(deg_ref, x_ref, w1_ref, t1_ref, g1_ref, dinv_ref, dinv2_ref):
    deg = deg_ref[0] + deg_ref[1] + 1.0
    dinv = jnp.where(deg > 0,
                     lax.rsqrt(jnp.maximum(deg, 1e-12)), 0.0)
    t = jnp.dot(x_ref[...], w1_ref[...],
                preferred_element_type=jnp.float32)
    t1_ref[...] = t
    g1_ref[...] = t * dinv
    dinv_ref[...] = dinv
    dinv2_ref[...] = dinv * dinv


def _tc_mid(accp_ref, t_ref, dinv_ref, dinv2_ref, b_ref, w_ref,
            tn_ref, gn_ref):
    a = (dinv_ref[...] * (accp_ref[0] + accp_ref[1])
         + dinv2_ref[...] * t_ref[...] + b_ref[...])
    h = jnp.maximum(a, 0.0)
    t = jnp.dot(h, w_ref[...], preferred_element_type=jnp.float32)
    tn_ref[...] = t
    gn_ref[...] = t * dinv_ref[...]


def _tc_pre_final(accp_ref, t_ref, dinv_ref, dinv2_ref, b_ref,
                  h_ref, gn_ref):
    a = (dinv_ref[...] * (accp_ref[0] + accp_ref[1])
         + dinv2_ref[...] * t_ref[...] + b_ref[...])
    h = jnp.maximum(a, 0.0)
    h_ref[...] = h
    gn_ref[...] = h * dinv_ref[...]


def _tc_final(accp_ref, h_ref, dinv_ref, dinv2_ref, b_ref, w_ref, out_ref):
    a = (dinv_ref[...] * (accp_ref[0] + accp_ref[1])
         + dinv2_ref[...] * h_ref[...])
    out_ref[...] = (jnp.dot(a, w_ref[...], preferred_element_type=jnp.float32)
                    + b_ref[...])


def _rows(block_shape):
    return pl.BlockSpec(block_shape, lambda i: (i,) + (0,) * (len(block_shape) - 1))


def _accp_spec():
    return pl.BlockSpec((NC, R, H), lambda i: (0, i, 0))


def _full(shape):
    return pl.BlockSpec(shape, lambda i: (0,) * len(shape))


def kernel(x, edge_index, edge_weight, W1, b1, W2, b2, W3, b3):
    pad = E_PAD - E
    src = jnp.concatenate([edge_index[0], jnp.zeros((pad,), jnp.int32)])
    dst = jnp.concatenate([edge_index[1], jnp.zeros((pad,), jnp.int32)])
    w = jnp.concatenate([edge_weight, jnp.zeros((pad,), jnp.float32)])
    src = src.reshape(NW * CPW, CHUNK)
    dst = dst.reshape(NW * CPW, CHUNK)
    w = w.reshape(NW * CPW, CHUNK)

    degp = _sc_degree(dst, w)
    degp3 = degp.reshape(NC, N_PAD, 1)

    b1r = b1.reshape(1, H)
    b2r = b2.reshape(1, H)
    b3r = b3.reshape(1, D_OUT)

    t1, g1, dinv, dinv2 = pl.pallas_call(
        _tc_first,
        grid=(G,),
        in_specs=[pl.BlockSpec((NC, R, 1), lambda i: (0, i, 0)),
                  _rows((R, D_IN)), _full((D_IN, H))],
        out_specs=[_rows((R, H)), _rows((R, H)),
                   _rows((R, 1)), _rows((R, 1))],
        out_shape=[jax.ShapeDtypeStruct((N, H), jnp.float32),
                   jax.ShapeDtypeStruct((N, H), jnp.float32),
                   jax.ShapeDtypeStruct((N, 1), jnp.float32),
                   jax.ShapeDtypeStruct((N, 1), jnp.float32)],
    )(degp3, x, W1)

    acc1 = _sc_aggregate(g1, src, dst, w)

    t2, g2 = pl.pallas_call(
        _tc_mid,
        grid=(G,),
        in_specs=[_accp_spec(), _rows((R, H)), _rows((R, 1)),
                  _rows((R, 1)), _full((1, H)), _full((H, H))],
        out_specs=[_rows((R, H)), _rows((R, H))],
        out_shape=[jax.ShapeDtypeStruct((N, H), jnp.float32),
                   jax.ShapeDtypeStruct((N, H), jnp.float32)],
    )(acc1, t1, dinv, dinv2, b1r, W2)

    acc2 = _sc_aggregate(g2, src, dst, w)

    h2, g3 = pl.pallas_call(
        _tc_pre_final,
        grid=(G,),
        in_specs=[_accp_spec(), _rows((R, H)), _rows((R, 1)),
                  _rows((R, 1)), _full((1, H))],
        out_specs=[_rows((R, H)), _rows((R, H))],
        out_shape=[jax.ShapeDtypeStruct((N, H), jnp.float32),
                   jax.ShapeDtypeStruct((N, H), jnp.float32)],
    )(acc2, t2, dinv, dinv2, b2r)

    acc3 = _sc_aggregate(g3, src, dst, w)

    out = pl.pallas_call(
        _tc_final,
        grid=(G,),
        in_specs=[_accp_spec(), _rows((R, H)), _rows((R, 1)),
                  _rows((R, 1)), _full((1, D_OUT)), _full((H, D_OUT))],
        out_specs=_rows((R, D_OUT)),
        out_shape=jax.ShapeDtypeStruct((N, D_OUT), jnp.float32),
    )(acc3, h2, dinv, dinv2, b3r, W3)

    return out


# speedup vs baseline: 13.5327x; 1.0848x over previous
import functools

import jax
import jax.numpy as jnp
from jax import lax
from jax.experimental import pallas as pl
from jax.experimental.pallas import tpu as pltpu
from jax.experimental.pallas import tpu_sc as plsc

N = 10000
E = 320000
D_IN = 128
H = 64
D_OUT = 128

NC = 2
NS = 16
NW = NC * NS
CHUNK = 128
CPW = 80
EPW = CHUNK * CPW
E_PAD = NW * EPW
N_PAD = 10240
ROWS_PER_S = N_PAD // NS

_mesh = plsc.VectorSubcoreMesh(core_axis_name="c", subcore_axis_name="s")


@functools.partial(
    pl.kernel,
    out_type=jax.ShapeDtypeStruct((NC, N_PAD), jnp.float32),
    mesh=_mesh,
    scratch_types=[
        pltpu.VMEM_SHARED((N_PAD,), jnp.float32),
        pltpu.VMEM((CPW, CHUNK), jnp.int32),
        pltpu.VMEM((CPW, CHUNK), jnp.float32),
        pltpu.VMEM((ROWS_PER_S,), jnp.float32),
    ],
    compiler_params=pltpu.CompilerParams(use_tc_tiling_on_sc=False),
)
def _sc_degree(dst_hbm, w_hbm, out_hbm, acc, dstb, wb, zbuf):
    c = lax.axis_index("c")
    s = lax.axis_index("s")
    wid = s * NC + c

    def _z(i, _):
        zbuf[pl.ds(i * 16, 16)] = jnp.zeros((16,), jnp.float32)
        return 0
    lax.fori_loop(0, ROWS_PER_S // 16, _z, 0)
    pltpu.sync_copy(zbuf, acc.at[pl.ds(s * ROWS_PER_S, ROWS_PER_S)])

    pltpu.sync_copy(dst_hbm.at[pl.ds(wid * CPW, CPW)], dstb)
    pltpu.sync_copy(w_hbm.at[pl.ds(wid * CPW, CPW)], wb)
    plsc.subcore_barrier()

    def _chunk(ci, _):
        pltpu.sync_copy(wb.at[ci], acc.at[dstb.at[ci]], add=True)
        return 0
    lax.fori_loop(0, CPW, _chunk, 0)

    plsc.subcore_barrier()
    pltpu.sync_copy(acc.at[pl.ds(s * ROWS_PER_S, ROWS_PER_S)],
                    out_hbm.at[c, pl.ds(s * ROWS_PER_S, ROWS_PER_S)])


@functools.partial(
    pl.kernel,
    out_type=jax.ShapeDtypeStruct((NC, N_PAD, H), jnp.float32),
    mesh=_mesh,
    scratch_types=[
        pltpu.VMEM_SHARED((N_PAD, H), jnp.float32),
        pltpu.VMEM((CPW, CHUNK), jnp.int32),
        pltpu.VMEM((CPW, CHUNK), jnp.int32),
        pltpu.VMEM((CPW, CHUNK), jnp.float32),
        [pltpu.VMEM((CHUNK, H), jnp.float32) for _ in range(4)],
        [pltpu.VMEM((CHUNK, H), jnp.float32) for _ in range(2)],
        [pltpu.SemaphoreType.DMA for _ in range(4)],
        [pltpu.SemaphoreType.DMA for _ in range(2)],
    ],
    compiler_params=pltpu.CompilerParams(use_tc_tiling_on_sc=False),
)
def _sc_aggregate(g_hbm, src_hbm, dst_hbm, w_hbm, out_hbm,
                  acc, srcb, dstb, wb, gbufs, sbufs, gsems, ssems):
    c = lax.axis_index("c")
    s = lax.axis_index("s")
    wid = s * NC + c

    def _z(i, _):
        for j in range(H // 16):
            gbufs[0][i, pl.ds(j * 16, 16)] = jnp.zeros((16,), jnp.float32)
        return 0
    lax.fori_loop(0, CHUNK, _z, 0)
    for k in range(ROWS_PER_S // CHUNK):
        pltpu.sync_copy(gbufs[0],
                        acc.at[pl.ds(s * ROWS_PER_S + k * CHUNK, CHUNK)])

    pltpu.sync_copy(src_hbm.at[pl.ds(wid * CPW, CPW)], srcb)
    pltpu.sync_copy(dst_hbm.at[pl.ds(wid * CPW, CPW)], dstb)
    pltpu.sync_copy(w_hbm.at[pl.ds(wid * CPW, CPW)], wb)
    plsc.subcore_barrier()

    for k in range(4):
        pltpu.async_copy(g_hbm.at[srcb.at[k]], gbufs[k], gsems[k])

    def _quad(i, _):
        for k in range(4):
            ci = 4 * i + k
            gb, gsem = gbufs[k], gsems[k]
            sb, ssem = sbufs[k % 2], ssems[k % 2]
            pltpu.make_async_copy(g_hbm.at[srcb.at[ci]], gb, gsem).wait()

            def _drain_sb():
                pltpu.make_async_copy(sb, acc.at[dstb.at[ci]], ssem).wait()
            if k >= 2:
                _drain_sb()
            else:
                pl.when(i > 0)(_drain_sb)

            def _scale(gi, _):
                wv = wb[ci, pl.ds(gi * 16, 16)]
                for el in range(16):
                    e = gi * 16 + el
                    we = wv[el]
                    for j in range(H // 16):
                        sl = pl.ds(j * 16, 16)
                        sb[e, sl] = gb[e, sl] * we
                return 0
            lax.fori_loop(0, CHUNK // 16, _scale, 0)

            pltpu.async_copy(sb, acc.at[dstb.at[ci]], ssem, add=True)

            @pl.when(ci + 4 < CPW)
            def _():
                pltpu.async_copy(g_hbm.at[srcb.at[ci + 4]], gb, gsem)
        return 0
    lax.fori_loop(0, CPW // 4, _quad, 0)

    for k in range(2):
        pltpu.make_async_copy(sbufs[k], acc.at[dstb.at[0]], ssems[k]).wait()

    plsc.subcore_barrier()
    pltpu.sync_copy(acc.at[pl.ds(s * ROWS_PER_S, ROWS_PER_S)],
                    out_hbm.at[c, pl.ds(s * ROWS_PER_S, ROWS_PER_S)])


R = 2000
G = N // R


def _tc_first(deg_ref, x_ref, w1_ref, t1_ref, g1_ref, dinv_ref, dinv2_ref):
    deg = deg_ref[0] + deg_ref[1] + 1.0
    dinv = jnp.where(deg > 0,
                     lax.rsqrt(jnp.maximum(deg, 1e-12)), 0.0)
    t = jnp.dot(x_ref[...], w1_ref[...],
                preferred_element_type=jnp.float32)
    t1_ref[...] = t
    g1_ref[...] = t * dinv
    dinv_ref[...] = dinv
    dinv2_ref[...] = dinv * dinv


def _tc_mid(accp_ref, t_ref, dinv_ref, dinv2_ref, b_ref, w_ref,
            tn_ref, gn_ref):
    a = (dinv_ref[...] * (accp_ref[0] + accp_ref[1])
         + dinv2_ref[...] * t_ref[...] + b_ref[...])
    h = jnp.maximum(a, 0.0)
    t = jnp.dot(h, w_ref[...], preferred_element_type=jnp.float32)
    tn_ref[...] = t
    gn_ref[...] = t * dinv_ref[...]


def _tc_pre_final(accp_ref, t_ref, dinv_ref, dinv2_ref, b_ref,
                  h_ref, gn_ref):
    a = (dinv_ref[...] * (accp_ref[0] + accp_ref[1])
         + dinv2_ref[...] * t_ref[...] + b_ref[...])
    h = jnp.maximum(a, 0.0)
    h_ref[...] = h
    gn_ref[...] = h * dinv_ref[...]


def _tc_final(accp_ref, h_ref, dinv_ref, dinv2_ref, b_ref, w_ref, out_ref):
    a = (dinv_ref[...] * (accp_ref[0] + accp_ref[1])
         + dinv2_ref[...] * h_ref[...])
    out_ref[...] = (jnp.dot(a, w_ref[...], preferred_element_type=jnp.float32)
                    + b_ref[...])


def _rows(block_shape):
    return pl.BlockSpec(block_shape, lambda i: (i,) + (0,) * (len(block_shape) - 1))


def _accp_spec():
    return pl.BlockSpec((NC, R, H), lambda i: (0, i, 0))


def _full(shape):
    return pl.BlockSpec(shape, lambda i: (0,) * len(shape))


def kernel(x, edge_index, edge_weight, W1, b1, W2, b2, W3, b3):
    pad = E_PAD - E
    src = jnp.concatenate([edge_index[0], jnp.zeros((pad,), jnp.int32)])
    dst = jnp.concatenate([edge_index[1], jnp.zeros((pad,), jnp.int32)])
    w = jnp.concatenate([edge_weight, jnp.zeros((pad,), jnp.float32)])
    src = src.reshape(NW * CPW, CHUNK)
    dst = dst.reshape(NW * CPW, CHUNK)
    w = w.reshape(NW * CPW, CHUNK)

    degp = _sc_degree(dst, w)
    degp3 = degp.reshape(NC, N_PAD, 1)

    b1r = b1.reshape(1, H)
    b2r = b2.reshape(1, H)
    b3r = b3.reshape(1, D_OUT)

    t1, g1, dinv, dinv2 = pl.pallas_call(
        _tc_first,
        grid=(G,),
        in_specs=[pl.BlockSpec((NC, R, 1), lambda i: (0, i, 0)),
                  _rows((R, D_IN)), _full((D_IN, H))],
        out_specs=[_rows((R, H)), _rows((R, H)),
                   _rows((R, 1)), _rows((R, 1))],
        out_shape=[jax.ShapeDtypeStruct((N, H), jnp.float32),
                   jax.ShapeDtypeStruct((N, H), jnp.float32),
                   jax.ShapeDtypeStruct((N, 1), jnp.float32),
                   jax.ShapeDtypeStruct((N, 1), jnp.float32)],
    )(degp3, x, W1)

    acc1 = _sc_aggregate(g1, src, dst, w)

    t2, g2 = pl.pallas_call(
        _tc_mid,
        grid=(G,),
        in_specs=[_accp_spec(), _rows((R, H)), _rows((R, 1)),
                  _rows((R, 1)), _full((1, H)), _full((H, H))],
        out_specs=[_rows((R, H)), _rows((R, H))],
        out_shape=[jax.ShapeDtypeStruct((N, H), jnp.float32),
                   jax.ShapeDtypeStruct((N, H), jnp.float32)],
    )(acc1, t1, dinv, dinv2, b1r, W2)

    acc2 = _sc_aggregate(g2, src, dst, w)

    h2, g3 = pl.pallas_call(
        _tc_pre_final,
        grid=(G,),
        in_specs=[_accp_spec(), _rows((R, H)), _rows((R, 1)),
                  _rows((R, 1)), _full((1, H))],
        out_specs=[_rows((R, H)), _rows((R, H))],
        out_shape=[jax.ShapeDtypeStruct((N, H), jnp.float32),
                   jax.ShapeDtypeStruct((N, H), jnp.float32)],
    )(acc2, t2, dinv, dinv2, b2r)

    acc3 = _sc_aggregate(g3, src, dst, w)

    out = pl.pallas_call(
        _tc_final,
        grid=(G,),
        in_specs=[_accp_spec(), _rows((R, H)), _rows((R, 1)),
                  _rows((R, 1)), _full((1, D_OUT)), _full((H, D_OUT))],
        out_specs=_rows((R, D_OUT)),
        out_shape=jax.ShapeDtypeStruct((N, D_OUT), jnp.float32),
    )(acc3, h2, dinv, dinv2, b3r, W3)

    return out


# speedup vs baseline: 19.2532x; 1.4227x over previous
import functools

import jax
import jax.numpy as jnp
from jax import lax
from jax.experimental import pallas as pl
from jax.experimental.pallas import tpu as pltpu
from jax.experimental.pallas import tpu_sc as plsc

N = 10000
E = 320000
D_IN = 128
H = 64
D_OUT = 128

NC = 2
NS = 16
NW = NC * NS
CHUNK = 128
CPW = 80
EPW = CHUNK * CPW
E_PAD = NW * EPW
N_PAD = 10240
ROWS_PER_S = N_PAD // NS

_mesh = plsc.VectorSubcoreMesh(core_axis_name="c", subcore_axis_name="s")


@functools.partial(
    pl.kernel,
    out_type=jax.ShapeDtypeStruct((NC, N_PAD), jnp.float32),
    mesh=_mesh,
    scratch_types=[
        pltpu.VMEM_SHARED((N_PAD,), jnp.float32),
        pltpu.VMEM((CPW, CHUNK), jnp.int32),
        pltpu.VMEM((CPW, CHUNK), jnp.float32),
        pltpu.VMEM((ROWS_PER_S,), jnp.float32),
    ],
    compiler_params=pltpu.CompilerParams(use_tc_tiling_on_sc=False),
)
def _sc_degree(dst_hbm, w_hbm, out_hbm, acc, dstb, wb, zbuf):
    c = lax.axis_index("c")
    s = lax.axis_index("s")
    wid = s * NC + c

    def _z(i, _):
        zbuf[pl.ds(i * 16, 16)] = jnp.zeros((16,), jnp.float32)
        return 0
    lax.fori_loop(0, ROWS_PER_S // 16, _z, 0)
    pltpu.sync_copy(zbuf, acc.at[pl.ds(s * ROWS_PER_S, ROWS_PER_S)])

    pltpu.sync_copy(dst_hbm.at[pl.ds(wid * CPW, CPW)], dstb)
    pltpu.sync_copy(w_hbm.at[pl.ds(wid * CPW, CPW)], wb)
    plsc.subcore_barrier()

    def _chunk(ci, _):
        pltpu.sync_copy(wb.at[ci], acc.at[dstb.at[ci]], add=True)
        return 0
    lax.fori_loop(0, CPW, _chunk, 0)

    plsc.subcore_barrier()
    pltpu.sync_copy(acc.at[pl.ds(s * ROWS_PER_S, ROWS_PER_S)],
                    out_hbm.at[c, pl.ds(s * ROWS_PER_S, ROWS_PER_S)])


@functools.partial(
    pl.kernel,
    out_type=jax.ShapeDtypeStruct((NC, N_PAD, H), jnp.float32),
    mesh=_mesh,
    scratch_types=[
        pltpu.VMEM_SHARED((N_PAD, H), jnp.float32),
        pltpu.VMEM((CPW, CHUNK), jnp.int32),
        pltpu.VMEM((CPW, CHUNK), jnp.int32),
        pltpu.VMEM((CPW, CHUNK), jnp.float32),
        [pltpu.VMEM((CHUNK, H), jnp.bfloat16) for _ in range(4)],
        [pltpu.VMEM((CHUNK, H), jnp.float32) for _ in range(2)],
        [pltpu.SemaphoreType.DMA for _ in range(4)],
        [pltpu.SemaphoreType.DMA for _ in range(2)],
    ],
    compiler_params=pltpu.CompilerParams(use_tc_tiling_on_sc=False,
                                         needs_layout_passes=False),
)
def _sc_aggregate(g_hbm, src_hbm, dst_hbm, w_hbm, out_hbm,
                  acc, srcb, dstb, wb, gbufs, sbufs, gsems, ssems):
    c = lax.axis_index("c")
    s = lax.axis_index("s")
    wid = s * NC + c

    def _z(i, _):
        for j in range(H // 16):
            sbufs[0][i, pl.ds(j * 16, 16)] = jnp.zeros((16,), jnp.float32)
        return 0
    lax.fori_loop(0, CHUNK, _z, 0)
    for k in range(ROWS_PER_S // CHUNK):
        pltpu.sync_copy(sbufs[0],
                        acc.at[pl.ds(s * ROWS_PER_S + k * CHUNK, CHUNK)])

    pltpu.sync_copy(src_hbm.at[pl.ds(wid * CPW, CPW)], srcb)
    pltpu.sync_copy(dst_hbm.at[pl.ds(wid * CPW, CPW)], dstb)
    pltpu.sync_copy(w_hbm.at[pl.ds(wid * CPW, CPW)], wb)
    plsc.subcore_barrier()

    for k in range(4):
        pltpu.async_copy(g_hbm.at[srcb.at[k]], gbufs[k], gsems[k])

    def _quad(i, _):
        for k in range(4):
            ci = 4 * i + k
            gb, gsem = gbufs[k], gsems[k]
            sb, ssem = sbufs[k % 2], ssems[k % 2]
            pltpu.make_async_copy(g_hbm.at[srcb.at[ci]], gb, gsem).wait()

            def _drain_sb():
                pltpu.make_async_copy(sb, acc.at[dstb.at[ci]], ssem).wait()
            if k >= 2:
                _drain_sb()
            else:
                pl.when(i > 0)(_drain_sb)

            def _scale(gi, _):
                wv = wb[ci, pl.ds(gi * 16, 16)]
                for el in range(16):
                    e = gi * 16 + el
                    we = wv[el]
                    for h2 in range(2):
                        v = gb[e, pl.ds(32 * h2, 32)]
                        lo, hi = plsc.unpack(
                            v, format=plsc.PackFormat.INTERLEAVED)
                        sb[e, pl.ds(32 * h2, 16)] = lo * we
                        sb[e, pl.ds(32 * h2 + 16, 16)] = hi * we
                return 0
            lax.fori_loop(0, CHUNK // 16, _scale, 0)

            pltpu.async_copy(sb, acc.at[dstb.at[ci]], ssem, add=True)

            @pl.when(ci + 4 < CPW)
            def _():
                pltpu.async_copy(g_hbm.at[srcb.at[ci + 4]], gb, gsem)
        return 0
    lax.fori_loop(0, CPW // 4, _quad, 0)

    for k in range(2):
        pltpu.make_async_copy(sbufs[k], acc.at[dstb.at[0]], ssems[k]).wait()

    plsc.subcore_barrier()
    pltpu.sync_copy(acc.at[pl.ds(s * ROWS_PER_S, ROWS_PER_S)],
                    out_hbm.at[c, pl.ds(s * ROWS_PER_S, ROWS_PER_S)])


R = 2000
G = N // R


def _tc_first(deg_ref, x_ref, w1_ref, t1_ref, g1_ref, dinv_ref, dinv2_ref):
    deg = deg_ref[0] + deg_ref[1] + 1.0
    dinv = jnp.where(deg > 0,
                     lax.rsqrt(jnp.maximum(deg, 1e-12)), 0.0)
    t = jnp.dot(x_ref[...], w1_ref[...],
                preferred_element_type=jnp.float32)
    t1_ref[...] = t
    g1_ref[...] = (t * dinv).astype(jnp.bfloat16)
    dinv_ref[...] = dinv
    dinv2_ref[...] = dinv * dinv


def _tc_mid(accp_ref, t_ref, dinv_ref, dinv2_ref, b_ref, w_ref,
            tn_ref, gn_ref):
    a = (dinv_ref[...] * (accp_ref[0] + accp_ref[1])
         + dinv2_ref[...] * t_ref[...] + b_ref[...])
    h = jnp.maximum(a, 0.0)
    t = jnp.dot(h, w_ref[...], preferred_element_type=jnp.float32)
    tn_ref[...] = t
    gn_ref[...] = (t * dinv_ref[...]).astype(jnp.bfloat16)


def _tc_pre_final(accp_ref, t_ref, dinv_ref, dinv2_ref, b_ref,
                  h_ref, gn_ref):
    a = (dinv_ref[...] * (accp_ref[0] + accp_ref[1])
         + dinv2_ref[...] * t_ref[...] + b_ref[...])
    h = jnp.maximum(a, 0.0)
    h_ref[...] = h
    gn_ref[...] = (h * dinv_ref[...]).astype(jnp.bfloat16)


def _tc_final(accp_ref, h_ref, dinv_ref, dinv2_ref, b_ref, w_ref, out_ref):
    a = (dinv_ref[...] * (accp_ref[0] + accp_ref[1])
         + dinv2_ref[...] * h_ref[...])
    out_ref[...] = (jnp.dot(a, w_ref[...], preferred_element_type=jnp.float32)
                    + b_ref[...])


def _rows(block_shape):
    return pl.BlockSpec(block_shape, lambda i: (i,) + (0,) * (len(block_shape) - 1))


def _accp_spec():
    return pl.BlockSpec((NC, R, H), lambda i: (0, i, 0))


def _full(shape):
    return pl.BlockSpec(shape, lambda i: (0,) * len(shape))


_PERM = [0] * H
for _h in (0, 1):
    for _k in range(16):
        _PERM[32 * _h + 2 * _k] = 32 * _h + _k
        _PERM[32 * _h + 2 * _k + 1] = 32 * _h + 16 + _k


def kernel(x, edge_index, edge_weight, W1, b1, W2, b2, W3, b3):
    pad = E_PAD - E
    src = jnp.concatenate([edge_index[0], jnp.zeros((pad,), jnp.int32)])
    dst = jnp.concatenate([edge_index[1], jnp.zeros((pad,), jnp.int32)])
    w = jnp.concatenate([edge_weight, jnp.zeros((pad,), jnp.float32)])
    src = src.reshape(NW * CPW, CHUNK)
    dst = dst.reshape(NW * CPW, CHUNK)
    w = w.reshape(NW * CPW, CHUNK)

    degp = _sc_degree(dst, w)
    degp3 = degp.reshape(NC, N_PAD, 1)

    b1r = b1.reshape(1, H)
    b2r = b2.reshape(1, H)
    b3r = b3.reshape(1, D_OUT)

    t1, g1, dinv, dinv2 = pl.pallas_call(
        _tc_first,
        grid=(G,),
        in_specs=[pl.BlockSpec((NC, R, 1), lambda i: (0, i, 0)),
                  _rows((R, D_IN)), _full((D_IN, H))],
        out_specs=[_rows((R, H)), _rows((R, H)),
                   _rows((R, 1)), _rows((R, 1))],
        out_shape=[jax.ShapeDtypeStruct((N, H), jnp.float32),
                   jax.ShapeDtypeStruct((N, H), jnp.bfloat16),
                   jax.ShapeDtypeStruct((N, 1), jnp.float32),
                   jax.ShapeDtypeStruct((N, 1), jnp.float32)],
    )(degp3, x, W1)

    perm = jnp.array(_PERM, dtype=jnp.int32)
    acc1 = _sc_aggregate(g1[:, perm], src, dst, w)

    t2, g2 = pl.pallas_call(
        _tc_mid,
        grid=(G,),
        in_specs=[_accp_spec(), _rows((R, H)), _rows((R, 1)),
                  _rows((R, 1)), _full((1, H)), _full((H, H))],
        out_specs=[_rows((R, H)), _rows((R, H))],
        out_shape=[jax.ShapeDtypeStruct((N, H), jnp.float32),
                   jax.ShapeDtypeStruct((N, H), jnp.bfloat16)],
    )(acc1, t1, dinv, dinv2, b1r, W2)

    acc2 = _sc_aggregate(g2[:, perm], src, dst, w)

    h2, g3 = pl.pallas_call(
        _tc_pre_final,
        grid=(G,),
        in_specs=[_accp_spec(), _rows((R, H)), _rows((R, 1)),
                  _rows((R, 1)), _full((1, H))],
        out_specs=[_rows((R, H)), _rows((R, H))],
        out_shape=[jax.ShapeDtypeStruct((N, H), jnp.float32),
                   jax.ShapeDtypeStruct((N, H), jnp.bfloat16)],
    )(acc2, t2, dinv, dinv2, b2r)

    acc3 = _sc_aggregate(g3[:, perm], src, dst, w)

    out = pl.pallas_call(
        _tc_final,
        grid=(G,),
        in_specs=[_accp_spec(), _rows((R, H)), _rows((R, 1)),
                  _rows((R, 1)), _full((1, D_OUT)), _full((H, D_OUT))],
        out_specs=_rows((R, D_OUT)),
        out_shape=jax.ShapeDtypeStruct((N, D_OUT), jnp.float32),
    )(acc3, h2, dinv, dinv2, b3r, W3)

    return out


# speedup vs baseline: 19.2560x; 1.0001x over previous
import functools

import jax
import jax.numpy as jnp
from jax import lax
from jax.experimental import pallas as pl
from jax.experimental.pallas import tpu as pltpu
from jax.experimental.pallas import tpu_sc as plsc

N = 10000
E = 320000
D_IN = 128
H = 64
D_OUT = 128

NC = 2
NS = 16
NW = NC * NS
CHUNK = 128
CPW = 80
EPW = CHUNK * CPW
E_PAD = NW * EPW
N_PAD = 10240
ROWS_PER_S = N_PAD // NS

_mesh = plsc.VectorSubcoreMesh(core_axis_name="c", subcore_axis_name="s")


@functools.partial(
    pl.kernel,
    out_type=jax.ShapeDtypeStruct((NC, N_PAD), jnp.float32),
    mesh=_mesh,
    scratch_types=[
        pltpu.VMEM_SHARED((N_PAD,), jnp.float32),
        pltpu.VMEM((CPW, CHUNK), jnp.int32),
        pltpu.VMEM((CPW, CHUNK), jnp.float32),
        pltpu.VMEM((ROWS_PER_S,), jnp.float32),
    ],
    compiler_params=pltpu.CompilerParams(use_tc_tiling_on_sc=False),
)
def _sc_degree(dst_hbm, w_hbm, out_hbm, acc, dstb, wb, zbuf):
    c = lax.axis_index("c")
    s = lax.axis_index("s")
    wid = s * NC + c

    def _z(i, _):
        zbuf[pl.ds(i * 16, 16)] = jnp.zeros((16,), jnp.float32)
        return 0
    lax.fori_loop(0, ROWS_PER_S // 16, _z, 0)
    pltpu.sync_copy(zbuf, acc.at[pl.ds(s * ROWS_PER_S, ROWS_PER_S)])

    pltpu.sync_copy(dst_hbm.at[pl.ds(wid * CPW, CPW)], dstb)
    pltpu.sync_copy(w_hbm.at[pl.ds(wid * CPW, CPW)], wb)
    plsc.subcore_barrier()

    def _chunk(ci, _):
        pltpu.sync_copy(wb.at[ci], acc.at[dstb.at[ci]], add=True)
        return 0
    lax.fori_loop(0, CPW, _chunk, 0)

    plsc.subcore_barrier()
    pltpu.sync_copy(acc.at[pl.ds(s * ROWS_PER_S, ROWS_PER_S)],
                    out_hbm.at[c, pl.ds(s * ROWS_PER_S, ROWS_PER_S)])


@functools.partial(
    pl.kernel,
    out_type=jax.ShapeDtypeStruct((NC, N_PAD, H), jnp.float32),
    mesh=_mesh,
    scratch_types=[
        pltpu.VMEM_SHARED((N_PAD, H), jnp.float32),
        pltpu.VMEM((CPW, CHUNK), jnp.int32),
        pltpu.VMEM((CPW, CHUNK), jnp.int32),
        pltpu.VMEM((CPW, CHUNK), jnp.float32),
        [pltpu.VMEM((CHUNK, H), jnp.bfloat16) for _ in range(4)],
        [pltpu.VMEM((CHUNK, H), jnp.float32) for _ in range(2)],
        [pltpu.SemaphoreType.DMA for _ in range(4)],
        [pltpu.SemaphoreType.DMA for _ in range(2)],
    ],
    compiler_params=pltpu.CompilerParams(use_tc_tiling_on_sc=False,
                                         needs_layout_passes=False),
)
def _sc_aggregate(g_hbm, src_hbm, dst_hbm, w_hbm, out_hbm,
                  acc, srcb, dstb, wb, gbufs, sbufs, gsems, ssems):
    c = lax.axis_index("c")
    s = lax.axis_index("s")
    wid = s * NC + c

    def _z(i, _):
        for j in range(H // 16):
            sbufs[0][i, pl.ds(j * 16, 16)] = jnp.zeros((16,), jnp.float32)
        return 0
    lax.fori_loop(0, CHUNK, _z, 0)
    for k in range(ROWS_PER_S // CHUNK):
        pltpu.sync_copy(sbufs[0],
                        acc.at[pl.ds(s * ROWS_PER_S + k * CHUNK, CHUNK)])

    pltpu.sync_copy(src_hbm.at[pl.ds(wid * CPW, CPW)], srcb)
    pltpu.sync_copy(dst_hbm.at[pl.ds(wid * CPW, CPW)], dstb)
    pltpu.sync_copy(w_hbm.at[pl.ds(wid * CPW, CPW)], wb)
    plsc.subcore_barrier()

    for k in range(4):
        pltpu.async_copy(g_hbm.at[srcb.at[k]], gbufs[k], gsems[k])

    def _quad(i, _):
        for k in range(4):
            ci = 4 * i + k
            gb, gsem = gbufs[k], gsems[k]
            sb, ssem = sbufs[k % 2], ssems[k % 2]
            pltpu.make_async_copy(g_hbm.at[srcb.at[ci]], gb, gsem).wait()

            def _drain_sb():
                pltpu.make_async_copy(sb, acc.at[dstb.at[ci]], ssem).wait()
            if k >= 2:
                _drain_sb()
            else:
                pl.when(i > 0)(_drain_sb)

            def _scale(gi, _):
                wv = wb[ci, pl.ds(gi * 16, 16)]
                for el in range(16):
                    e = gi * 16 + el
                    we = wv[el]
                    for h2 in range(2):
                        v = gb[e, pl.ds(32 * h2, 32)]
                        u = plsc.bitcast(v, jnp.uint32)
                        lo = plsc.bitcast(u << 16, jnp.float32)
                        hi = plsc.bitcast(u & jnp.uint32(0xFFFF0000),
                                          jnp.float32)
                        sb[e, pl.ds(32 * h2, 16)] = lo * we
                        sb[e, pl.ds(32 * h2 + 16, 16)] = hi * we
                return 0
            lax.fori_loop(0, CHUNK // 16, _scale, 0)

            pltpu.async_copy(sb, acc.at[dstb.at[ci]], ssem, add=True)

            @pl.when(ci + 4 < CPW)
            def _():
                pltpu.async_copy(g_hbm.at[srcb.at[ci + 4]], gb, gsem)
        return 0
    lax.fori_loop(0, CPW // 4, _quad, 0)

    for k in range(2):
        pltpu.make_async_copy(sbufs[k], acc.at[dstb.at[0]], ssems[k]).wait()

    plsc.subcore_barrier()
    pltpu.sync_copy(acc.at[pl.ds(s * ROWS_PER_S, ROWS_PER_S)],
                    out_hbm.at[c, pl.ds(s * ROWS_PER_S, ROWS_PER_S)])


R = 2000
G = N // R


def _tc_first(deg_ref, x_ref, w1_ref, t1_ref, g1_ref, dinv_ref, dinv2_ref):
    deg = deg_ref[0] + deg_ref[1] + 1.0
    dinv = jnp.where(deg > 0,
                     lax.rsqrt(jnp.maximum(deg, 1e-12)), 0.0)
    t = jnp.dot(x_ref[...], w1_ref[...],
                preferred_element_type=jnp.float32)
    t1_ref[...] = t
    g1_ref[...] = (t * dinv).astype(jnp.bfloat16)
    dinv_ref[...] = dinv
    dinv2_ref[...] = dinv * dinv


def _tc_mid(accp_ref, t_ref, dinv_ref, dinv2_ref, b_ref, w_ref,
            tn_ref, gn_ref):
    a = (dinv_ref[...] * (accp_ref[0] + accp_ref[1])
         + dinv2_ref[...] * t_ref[...] + b_ref[...])
    h = jnp.maximum(a, 0.0)
    t = jnp.dot(h, w_ref[...], preferred_element_type=jnp.float32)
    tn_ref[...] = t
    gn_ref[...] = (t * dinv_ref[...]).astype(jnp.bfloat16)


def _tc_pre_final(accp_ref, t_ref, dinv_ref, dinv2_ref, b_ref,
                  h_ref, gn_ref):
    a = (dinv_ref[...] * (accp_ref[0] + accp_ref[1])
         + dinv2_ref[...] * t_ref[...] + b_ref[...])
    h = jnp.maximum(a, 0.0)
    h_ref[...] = h
    gn_ref[...] = (h * dinv_ref[...]).astype(jnp.bfloat16)


def _tc_final(accp_ref, h_ref, dinv_ref, dinv2_ref, b_ref, w_ref, out_ref):
    a = (dinv_ref[...] * (accp_ref[0] + accp_ref[1])
         + dinv2_ref[...] * h_ref[...])
    out_ref[...] = (jnp.dot(a, w_ref[...], preferred_element_type=jnp.float32)
                    + b_ref[...])


def _rows(block_shape):
    return pl.BlockSpec(block_shape, lambda i: (i,) + (0,) * (len(block_shape) - 1))


def _accp_spec():
    return pl.BlockSpec((NC, R, H), lambda i: (0, i, 0))


def _full(shape):
    return pl.BlockSpec(shape, lambda i: (0,) * len(shape))


_PERM = [0] * H
for _h in (0, 1):
    for _k in range(16):
        _PERM[32 * _h + 2 * _k] = 32 * _h + _k
        _PERM[32 * _h + 2 * _k + 1] = 32 * _h + 16 + _k


def kernel(x, edge_index, edge_weight, W1, b1, W2, b2, W3, b3):
    pad = E_PAD - E
    src = jnp.concatenate([edge_index[0], jnp.zeros((pad,), jnp.int32)])
    dst = jnp.concatenate([edge_index[1], jnp.zeros((pad,), jnp.int32)])
    w = jnp.concatenate([edge_weight, jnp.zeros((pad,), jnp.float32)])
    src = src.reshape(NW * CPW, CHUNK)
    dst = dst.reshape(NW * CPW, CHUNK)
    w = w.reshape(NW * CPW, CHUNK)

    degp = _sc_degree(dst, w)
    degp3 = degp.reshape(NC, N_PAD, 1)

    b1r = b1.reshape(1, H)
    b2r = b2.reshape(1, H)
    b3r = b3.reshape(1, D_OUT)

    t1, g1, dinv, dinv2 = pl.pallas_call(
        _tc_first,
        grid=(G,),
        in_specs=[pl.BlockSpec((NC, R, 1), lambda i: (0, i, 0)),
                  _rows((R, D_IN)), _full((D_IN, H))],
        out_specs=[_rows((R, H)), _rows((R, H)),
                   _rows((R, 1)), _rows((R, 1))],
        out_shape=[jax.ShapeDtypeStruct((N, H), jnp.float32),
                   jax.ShapeDtypeStruct((N, H), jnp.bfloat16),
                   jax.ShapeDtypeStruct((N, 1), jnp.float32),
                   jax.ShapeDtypeStruct((N, 1), jnp.float32)],
    )(degp3, x, W1)

    perm = jnp.array(_PERM, dtype=jnp.int32)
    acc1 = _sc_aggregate(g1[:, perm], src, dst, w)

    t2, g2 = pl.pallas_call(
        _tc_mid,
        grid=(G,),
        in_specs=[_accp_spec(), _rows((R, H)), _rows((R, 1)),
                  _rows((R, 1)), _full((1, H)), _full((H, H))],
        out_specs=[_rows((R, H)), _rows((R, H))],
        out_shape=[jax.ShapeDtypeStruct((N, H), jnp.float32),
                   jax.ShapeDtypeStruct((N, H), jnp.bfloat16)],
    )(acc1, t1, dinv, dinv2, b1r, W2)

    acc2 = _sc_aggregate(g2[:, perm], src, dst, w)

    h2, g3 = pl.pallas_call(
        _tc_pre_final,
        grid=(G,),
        in_specs=[_accp_spec(), _rows((R, H)), _rows((R, 1)),
                  _rows((R, 1)), _full((1, H))],
        out_specs=[_rows((R, H)), _rows((R, H))],
        out_shape=[jax.ShapeDtypeStruct((N, H), jnp.float32),
                   jax.ShapeDtypeStruct((N, H), jnp.bfloat16)],
    )(acc2, t2, dinv, dinv2, b2r)

    acc3 = _sc_aggregate(g3[:, perm], src, dst, w)

    out = pl.pallas_call(
        _tc_final,
        grid=(G,),
        in_specs=[_accp_spec(), _rows((R, H)), _rows((R, 1)),
                  _rows((R, 1)), _full((1, D_OUT)), _full((H, D_OUT))],
        out_specs=_rows((R, D_OUT)),
        out_shape=jax.ShapeDtypeStruct((N, D_OUT), jnp.float32),
    )(acc3, h2, dinv, dinv2, b3r, W3)

    return out


# speedup vs baseline: 20.6783x; 1.0739x over previous
import functools

import jax
import jax.numpy as jnp
from jax import lax
from jax.experimental import pallas as pl
from jax.experimental.pallas import tpu as pltpu
from jax.experimental.pallas import tpu_sc as plsc

N = 10000
E = 320000
D_IN = 128
H = 64
D_OUT = 128

NC = 2
NS = 16
NW = NC * NS
CHUNK = 128
CPW = 80
EPW = CHUNK * CPW
E_PAD = NW * EPW
N_PAD = 10240
ROWS_PER_S = N_PAD // NS

_mesh = plsc.VectorSubcoreMesh(core_axis_name="c", subcore_axis_name="s")


@functools.partial(
    pl.kernel,
    out_type=jax.ShapeDtypeStruct((NC, N_PAD), jnp.float32),
    mesh=_mesh,
    scratch_types=[
        pltpu.VMEM_SHARED((N_PAD,), jnp.float32),
        pltpu.VMEM((CPW, CHUNK), jnp.int32),
        pltpu.VMEM((CPW, CHUNK), jnp.float32),
        pltpu.VMEM((ROWS_PER_S,), jnp.float32),
    ],
    compiler_params=pltpu.CompilerParams(use_tc_tiling_on_sc=False),
)
def _sc_degree(dst_hbm, w_hbm, out_hbm, acc, dstb, wb, zbuf):
    c = lax.axis_index("c")
    s = lax.axis_index("s")
    wid = s * NC + c

    def _z(i, _):
        zbuf[pl.ds(i * 16, 16)] = jnp.zeros((16,), jnp.float32)
        return 0
    lax.fori_loop(0, ROWS_PER_S // 16, _z, 0)
    pltpu.sync_copy(zbuf, acc.at[pl.ds(s * ROWS_PER_S, ROWS_PER_S)])

    pltpu.sync_copy(dst_hbm.at[pl.ds(wid * CPW, CPW)], dstb)
    pltpu.sync_copy(w_hbm.at[pl.ds(wid * CPW, CPW)], wb)
    plsc.subcore_barrier()

    def _chunk(ci, _):
        pltpu.sync_copy(wb.at[ci], acc.at[dstb.at[ci]], add=True)
        return 0
    lax.fori_loop(0, CPW, _chunk, 0)

    plsc.subcore_barrier()
    pltpu.sync_copy(acc.at[pl.ds(s * ROWS_PER_S, ROWS_PER_S)],
                    out_hbm.at[c, pl.ds(s * ROWS_PER_S, ROWS_PER_S)])


@functools.partial(
    pl.kernel,
    out_type=jax.ShapeDtypeStruct((NC, N_PAD, H), jnp.float32),
    mesh=_mesh,
    scratch_types=[
        pltpu.VMEM_SHARED((N_PAD, H), jnp.float32),
        pltpu.VMEM((CPW, CHUNK), jnp.int32),
        pltpu.VMEM((CPW, CHUNK), jnp.int32),
        pltpu.VMEM((CPW, CHUNK), jnp.float32),
        [pltpu.VMEM((CHUNK, H), jnp.bfloat16) for _ in range(4)],
        [pltpu.VMEM((CHUNK, H), jnp.float32) for _ in range(2)],
        [pltpu.SemaphoreType.DMA for _ in range(4)],
        [pltpu.SemaphoreType.DMA for _ in range(2)],
    ],
    compiler_params=pltpu.CompilerParams(use_tc_tiling_on_sc=False,
                                         needs_layout_passes=False),
)
def _sc_aggregate(g_hbm, src_hbm, dst_hbm, w_hbm, out_hbm,
                  acc, srcb, dstb, wb, gbufs, sbufs, gsems, ssems):
    c = lax.axis_index("c")
    s = lax.axis_index("s")
    wid = s * NC + c

    def _z(i, _):
        for j in range(H // 16):
            sbufs[0][i, pl.ds(j * 16, 16)] = jnp.zeros((16,), jnp.float32)
        return 0
    lax.fori_loop(0, CHUNK, _z, 0)
    for k in range(ROWS_PER_S // CHUNK):
        pltpu.sync_copy(sbufs[0],
                        acc.at[pl.ds(s * ROWS_PER_S + k * CHUNK, CHUNK)])

    pltpu.sync_copy(src_hbm.at[pl.ds(wid * CPW, CPW)], srcb)
    pltpu.sync_copy(dst_hbm.at[pl.ds(wid * CPW, CPW)], dstb)
    pltpu.sync_copy(w_hbm.at[pl.ds(wid * CPW, CPW)], wb)
    plsc.subcore_barrier()

    for k in range(4):
        pltpu.async_copy(g_hbm.at[srcb.at[k]], gbufs[k], gsems[k])

    def _quad(i, _):
        for k in range(4):
            ci = 4 * i + k
            gb, gsem = gbufs[k], gsems[k]
            sb, ssem = sbufs[k % 2], ssems[k % 2]
            pltpu.make_async_copy(g_hbm.at[srcb.at[ci]], gb, gsem).wait()

            def _drain_sb():
                pltpu.make_async_copy(sb, acc.at[dstb.at[ci]], ssem).wait()
            if k >= 2:
                _drain_sb()
            else:
                pl.when(i > 0)(_drain_sb)

            def _scale(gi, _):
                wv = wb[ci, pl.ds(gi * 16, 16)]
                for el in range(16):
                    e = gi * 16 + el
                    we = wv[el]
                    for h2 in range(2):
                        v = gb[e, pl.ds(32 * h2, 32)]
                        u = plsc.bitcast(v, jnp.uint32)
                        lo = plsc.bitcast(u << 16, jnp.float32)
                        hi = plsc.bitcast(u & jnp.uint32(0xFFFF0000),
                                          jnp.float32)
                        sb[e, pl.ds(32 * h2, 16)] = lo * we
                        sb[e, pl.ds(32 * h2 + 16, 16)] = hi * we
                return 0
            lax.fori_loop(0, CHUNK // 16, _scale, 0)

            pltpu.async_copy(sb, acc.at[dstb.at[ci]], ssem, add=True)

            @pl.when(ci + 4 < CPW)
            def _():
                pltpu.async_copy(g_hbm.at[srcb.at[ci + 4]], gb, gsem)
        return 0
    lax.fori_loop(0, CPW // 4, _quad, 0)

    for k in range(2):
        pltpu.make_async_copy(sbufs[k], acc.at[dstb.at[0]], ssems[k]).wait()

    plsc.subcore_barrier()
    pltpu.sync_copy(acc.at[pl.ds(s * ROWS_PER_S, ROWS_PER_S)],
                    out_hbm.at[c, pl.ds(s * ROWS_PER_S, ROWS_PER_S)])


R = 2000
G = N // R


def _tc_first(deg_ref, x_ref, w1_ref, p_ref, t1_ref, g1_ref,
              dinv_ref, dinv2_ref):
    deg = deg_ref[0] + deg_ref[1] + 1.0
    dinv = jnp.where(deg > 0,
                     lax.rsqrt(jnp.maximum(deg, 1e-12)), 0.0)
    t = jnp.dot(x_ref[...], w1_ref[...],
                preferred_element_type=jnp.float32)
    t1_ref[...] = t
    tp = jnp.dot(t, p_ref[...], preferred_element_type=jnp.float32)
    g1_ref[...] = (tp * dinv).astype(jnp.bfloat16)
    dinv_ref[...] = dinv
    dinv2_ref[...] = dinv * dinv


def _tc_mid(accp_ref, t_ref, dinv_ref, dinv2_ref, b_ref, w_ref, p_ref,
            tn_ref, gn_ref):
    a = (dinv_ref[...] * (accp_ref[0] + accp_ref[1])
         + dinv2_ref[...] * t_ref[...] + b_ref[...])
    h = jnp.maximum(a, 0.0)
    t = jnp.dot(h, w_ref[...], preferred_element_type=jnp.float32)
    tn_ref[...] = t
    tp = jnp.dot(t, p_ref[...], preferred_element_type=jnp.float32)
    gn_ref[...] = (tp * dinv_ref[...]).astype(jnp.bfloat16)


def _tc_pre_final(accp_ref, t_ref, dinv_ref, dinv2_ref, b_ref, p_ref,
                  h_ref, gn_ref):
    a = (dinv_ref[...] * (accp_ref[0] + accp_ref[1])
         + dinv2_ref[...] * t_ref[...] + b_ref[...])
    h = jnp.maximum(a, 0.0)
    h_ref[...] = h
    hp = jnp.dot(h, p_ref[...], preferred_element_type=jnp.float32)
    gn_ref[...] = (hp * dinv_ref[...]).astype(jnp.bfloat16)


def _tc_final(accp_ref, h_ref, dinv_ref, dinv2_ref, b_ref, w_ref, out_ref):
    a = (dinv_ref[...] * (accp_ref[0] + accp_ref[1])
         + dinv2_ref[...] * h_ref[...])
    out_ref[...] = (jnp.dot(a, w_ref[...], preferred_element_type=jnp.float32)
                    + b_ref[...])


def _rows(block_shape):
    return pl.BlockSpec(block_shape, lambda i: (i,) + (0,) * (len(block_shape) - 1))


def _accp_spec():
    return pl.BlockSpec((NC, R, H), lambda i: (0, i, 0))


def _full(shape):
    return pl.BlockSpec(shape, lambda i: (0,) * len(shape))


_PERM = [0] * H
for _h in (0, 1):
    for _k in range(16):
        _PERM[32 * _h + 2 * _k] = 32 * _h + _k
        _PERM[32 * _h + 2 * _k + 1] = 32 * _h + 16 + _k


def kernel(x, edge_index, edge_weight, W1, b1, W2, b2, W3, b3):
    pad = E_PAD - E
    src = jnp.concatenate([edge_index[0], jnp.zeros((pad,), jnp.int32)])
    dst = jnp.concatenate([edge_index[1], jnp.zeros((pad,), jnp.int32)])
    w = jnp.concatenate([edge_weight, jnp.zeros((pad,), jnp.float32)])
    src = src.reshape(NW * CPW, CHUNK)
    dst = dst.reshape(NW * CPW, CHUNK)
    w = w.reshape(NW * CPW, CHUNK)

    degp = _sc_degree(dst, w)
    degp3 = degp.reshape(NC, N_PAD, 1)

    b1r = b1.reshape(1, H)
    b2r = b2.reshape(1, H)
    b3r = b3.reshape(1, D_OUT)

    pmat = jnp.eye(H, dtype=jnp.float32)[jnp.array(_PERM, jnp.int32)].T

    t1, g1, dinv, dinv2 = pl.pallas_call(
        _tc_first,
        grid=(G,),
        in_specs=[pl.BlockSpec((NC, R, 1), lambda i: (0, i, 0)),
                  _rows((R, D_IN)), _full((D_IN, H)), _full((H, H))],
        out_specs=[_rows((R, H)), _rows((R, H)),
                   _rows((R, 1)), _rows((R, 1))],
        out_shape=[jax.ShapeDtypeStruct((N, H), jnp.float32),
                   jax.ShapeDtypeStruct((N, H), jnp.bfloat16),
                   jax.ShapeDtypeStruct((N, 1), jnp.float32),
                   jax.ShapeDtypeStruct((N, 1), jnp.float32)],
    )(degp3, x, W1, pmat)

    acc1 = _sc_aggregate(g1, src, dst, w)

    t2, g2 = pl.pallas_call(
        _tc_mid,
        grid=(G,),
        in_specs=[_accp_spec(), _rows((R, H)), _rows((R, 1)),
                  _rows((R, 1)), _full((1, H)), _full((H, H)),
                  _full((H, H))],
        out_specs=[_rows((R, H)), _rows((R, H))],
        out_shape=[jax.ShapeDtypeStruct((N, H), jnp.float32),
                   jax.ShapeDtypeStruct((N, H), jnp.bfloat16)],
    )(acc1, t1, dinv, dinv2, b1r, W2, pmat)

    acc2 = _sc_aggregate(g2, src, dst, w)

    h2, g3 = pl.pallas_call(
        _tc_pre_final,
        grid=(G,),
        in_specs=[_accp_spec(), _rows((R, H)), _rows((R, 1)),
                  _rows((R, 1)), _full((1, H)), _full((H, H))],
        out_specs=[_rows((R, H)), _rows((R, H))],
        out_shape=[jax.ShapeDtypeStruct((N, H), jnp.float32),
                   jax.ShapeDtypeStruct((N, H), jnp.bfloat16)],
    )(acc2, t2, dinv, dinv2, b2r, pmat)

    acc3 = _sc_aggregate(g3, src, dst, w)

    out = pl.pallas_call(
        _tc_final,
        grid=(G,),
        in_specs=[_accp_spec(), _rows((R, H)), _rows((R, 1)),
                  _rows((R, 1)), _full((1, D_OUT)), _full((H, D_OUT))],
        out_specs=_rows((R, D_OUT)),
        out_shape=jax.ShapeDtypeStruct((N, D_OUT), jnp.float32),
    )(acc3, h2, dinv, dinv2, b3r, W3)

    return out


# speedup vs baseline: 20.9296x; 1.0122x over previous
import functools

import jax
import jax.numpy as jnp
from jax import lax
from jax.experimental import pallas as pl
from jax.experimental.pallas import tpu as pltpu
from jax.experimental.pallas import tpu_sc as plsc

N = 10000
E = 320000
D_IN = 128
H = 64
D_OUT = 128

NC = 2
NS = 16
NW = NC * NS
CHUNK = 128
CPW = 80
EPW = CHUNK * CPW
E_PAD = NW * EPW
N_PAD = 10240
ROWS_PER_S = N_PAD // NS

_mesh = plsc.VectorSubcoreMesh(core_axis_name="c", subcore_axis_name="s")


@functools.partial(
    pl.kernel,
    out_type=jax.ShapeDtypeStruct((NC, N_PAD), jnp.float32),
    mesh=_mesh,
    scratch_types=[
        pltpu.VMEM_SHARED((N_PAD,), jnp.float32),
        pltpu.VMEM((CPW, CHUNK), jnp.int32),
        pltpu.VMEM((CPW, CHUNK), jnp.float32),
        pltpu.VMEM((ROWS_PER_S,), jnp.float32),
        pltpu.SemaphoreType.DMA,
    ],
    compiler_params=pltpu.CompilerParams(use_tc_tiling_on_sc=False),
)
def _sc_degree(dst_hbm, w_hbm, out_hbm, acc, dstb, wb, zbuf, sem):
    c = lax.axis_index("c")
    s = lax.axis_index("s")
    wid = s * NC + c

    def _z(i, _):
        zbuf[pl.ds(i * 16, 16)] = jnp.zeros((16,), jnp.float32)
        return 0
    lax.fori_loop(0, ROWS_PER_S // 16, _z, 0)
    pltpu.sync_copy(zbuf, acc.at[pl.ds(s * ROWS_PER_S, ROWS_PER_S)])

    pltpu.sync_copy(dst_hbm.at[pl.ds(wid * CPW, CPW)], dstb)
    pltpu.sync_copy(w_hbm.at[pl.ds(wid * CPW, CPW)], wb)
    plsc.subcore_barrier()

    def _chunk(ci, _):
        pltpu.async_copy(wb.at[ci], acc.at[dstb.at[ci]], sem, add=True)
        return 0
    lax.fori_loop(0, CPW, _chunk, 0)

    def _drain(ci, _):
        pltpu.make_async_copy(wb.at[ci], acc.at[dstb.at[ci]], sem).wait()
        return 0
    lax.fori_loop(0, CPW, _drain, 0)

    plsc.subcore_barrier()
    pltpu.sync_copy(acc.at[pl.ds(s * ROWS_PER_S, ROWS_PER_S)],
                    out_hbm.at[c, pl.ds(s * ROWS_PER_S, ROWS_PER_S)])


@functools.partial(
    pl.kernel,
    out_type=jax.ShapeDtypeStruct((NC, N_PAD, H), jnp.float32),
    mesh=_mesh,
    scratch_types=[
        pltpu.VMEM_SHARED((N_PAD, H), jnp.float32),
        pltpu.VMEM((CPW, CHUNK), jnp.int32),
        pltpu.VMEM((CPW, CHUNK), jnp.int32),
        pltpu.VMEM((CPW, CHUNK), jnp.float32),
        [pltpu.VMEM((CHUNK, H), jnp.bfloat16) for _ in range(4)],
        [pltpu.VMEM((CHUNK, H), jnp.float32) for _ in range(2)],
        [pltpu.SemaphoreType.DMA for _ in range(4)],
        [pltpu.SemaphoreType.DMA for _ in range(2)],
    ],
    compiler_params=pltpu.CompilerParams(use_tc_tiling_on_sc=False,
                                         needs_layout_passes=False),
)
def _sc_aggregate(g_hbm, src_hbm, dst_hbm, w_hbm, out_hbm,
                  acc, srcb, dstb, wb, gbufs, sbufs, gsems, ssems):
    c = lax.axis_index("c")
    s = lax.axis_index("s")
    wid = s * NC + c

    def _z(i, _):
        for j in range(H // 16):
            sbufs[0][i, pl.ds(j * 16, 16)] = jnp.zeros((16,), jnp.float32)
        return 0
    lax.fori_loop(0, CHUNK, _z, 0)
    for k in range(ROWS_PER_S // CHUNK):
        pltpu.sync_copy(sbufs[0],
                        acc.at[pl.ds(s * ROWS_PER_S + k * CHUNK, CHUNK)])

    pltpu.sync_copy(src_hbm.at[pl.ds(wid * CPW, CPW)], srcb)
    pltpu.sync_copy(dst_hbm.at[pl.ds(wid * CPW, CPW)], dstb)
    pltpu.sync_copy(w_hbm.at[pl.ds(wid * CPW, CPW)], wb)
    plsc.subcore_barrier()

    for k in range(4):
        pltpu.async_copy(g_hbm.at[srcb.at[k]], gbufs[k], gsems[k])

    def _quad(i, _):
        for k in range(4):
            ci = 4 * i + k
            gb, gsem = gbufs[k], gsems[k]
            sb, ssem = sbufs[k % 2], ssems[k % 2]
            pltpu.make_async_copy(g_hbm.at[srcb.at[ci]], gb, gsem).wait()

            def _drain_sb():
                pltpu.make_async_copy(sb, acc.at[dstb.at[ci]], ssem).wait()
            if k >= 2:
                _drain_sb()
            else:
                pl.when(i > 0)(_drain_sb)

            def _scale(gi, _):
                wv = wb[ci, pl.ds(gi * 16, 16)]
                for el in range(16):
                    e = gi * 16 + el
                    we = wv[el]
                    for h2 in range(2):
                        v = gb[e, pl.ds(32 * h2, 32)]
                        u = plsc.bitcast(v, jnp.uint32)
                        lo = plsc.bitcast(u << 16, jnp.float32)
                        hi = plsc.bitcast(u & jnp.uint32(0xFFFF0000),
                                          jnp.float32)
                        sb[e, pl.ds(32 * h2, 16)] = lo * we
                        sb[e, pl.ds(32 * h2 + 16, 16)] = hi * we
                return 0
            lax.fori_loop(0, CHUNK // 16, _scale, 0)

            pltpu.async_copy(sb, acc.at[dstb.at[ci]], ssem, add=True)

            @pl.when(ci + 4 < CPW)
            def _():
                pltpu.async_copy(g_hbm.at[srcb.at[ci + 4]], gb, gsem)
        return 0
    lax.fori_loop(0, CPW // 4, _quad, 0)

    for k in range(2):
        pltpu.make_async_copy(sbufs[k], acc.at[dstb.at[0]], ssems[k]).wait()

    plsc.subcore_barrier()
    pltpu.sync_copy(acc.at[pl.ds(s * ROWS_PER_S, ROWS_PER_S)],
                    out_hbm.at[c, pl.ds(s * ROWS_PER_S, ROWS_PER_S)])


R = 5000
G = N // R


def _tc_first(deg_ref, x_ref, w1_ref, p_ref, t1_ref, g1_ref,
              dinv_ref, dinv2_ref):
    deg = deg_ref[0] + deg_ref[1] + 1.0
    dinv = jnp.where(deg > 0,
                     lax.rsqrt(jnp.maximum(deg, 1e-12)), 0.0)
    t = jnp.dot(x_ref[...], w1_ref[...],
                preferred_element_type=jnp.float32)
    t1_ref[...] = t
    tp = jnp.dot(t, p_ref[...], preferred_element_type=jnp.float32)
    g1_ref[...] = (tp * dinv).astype(jnp.bfloat16)
    dinv_ref[...] = dinv
    dinv2_ref[...] = dinv * dinv


def _tc_mid(accp_ref, t_ref, dinv_ref, dinv2_ref, b_ref, w_ref, p_ref,
            tn_ref, gn_ref):
    a = (dinv_ref[...] * (accp_ref[0] + accp_ref[1])
         + dinv2_ref[...] * t_ref[...] + b_ref[...])
    h = jnp.maximum(a, 0.0)
    t = jnp.dot(h, w_ref[...], preferred_element_type=jnp.float32)
    tn_ref[...] = t
    tp = jnp.dot(t, p_ref[...], preferred_element_type=jnp.float32)
    gn_ref[...] = (tp * dinv_ref[...]).astype(jnp.bfloat16)


def _tc_pre_final(accp_ref, t_ref, dinv_ref, dinv2_ref, b_ref, p_ref,
                  h_ref, gn_ref):
    a = (dinv_ref[...] * (accp_ref[0] + accp_ref[1])
         + dinv2_ref[...] * t_ref[...] + b_ref[...])
    h = jnp.maximum(a, 0.0)
    h_ref[...] = h
    hp = jnp.dot(h, p_ref[...], preferred_element_type=jnp.float32)
    gn_ref[...] = (hp * dinv_ref[...]).astype(jnp.bfloat16)


def _tc_final(accp_ref, h_ref, dinv_ref, dinv2_ref, b_ref, w_ref, out_ref):
    a = (dinv_ref[...] * (accp_ref[0] + accp_ref[1])
         + dinv2_ref[...] * h_ref[...])
    out_ref[...] = (jnp.dot(a, w_ref[...], preferred_element_type=jnp.float32)
                    + b_ref[...])


def _rows(block_shape):
    return pl.BlockSpec(block_shape, lambda i: (i,) + (0,) * (len(block_shape) - 1))


def _accp_spec():
    return pl.BlockSpec((NC, R, H), lambda i: (0, i, 0))


def _full(shape):
    return pl.BlockSpec(shape, lambda i: (0,) * len(shape))


_PERM = [0] * H
for _h in (0, 1):
    for _k in range(16):
        _PERM[32 * _h + 2 * _k] = 32 * _h + _k
        _PERM[32 * _h + 2 * _k + 1] = 32 * _h + 16 + _k


def kernel(x, edge_index, edge_weight, W1, b1, W2, b2, W3, b3):
    pad = E_PAD - E
    src = jnp.concatenate([edge_index[0], jnp.zeros((pad,), jnp.int32)])
    dst = jnp.concatenate([edge_index[1], jnp.zeros((pad,), jnp.int32)])
    w = jnp.concatenate([edge_weight, jnp.zeros((pad,), jnp.float32)])
    src = src.reshape(NW * CPW, CHUNK)
    dst = dst.reshape(NW * CPW, CHUNK)
    w = w.reshape(NW * CPW, CHUNK)

    degp = _sc_degree(dst, w)
    degp3 = degp.reshape(NC, N_PAD, 1)

    b1r = b1.reshape(1, H)
    b2r = b2.reshape(1, H)
    b3r = b3.reshape(1, D_OUT)

    pmat = jnp.eye(H, dtype=jnp.float32)[jnp.array(_PERM, jnp.int32)].T

    t1, g1, dinv, dinv2 = pl.pallas_call(
        _tc_first,
        grid=(G,),
        in_specs=[pl.BlockSpec((NC, R, 1), lambda i: (0, i, 0)),
                  _rows((R, D_IN)), _full((D_IN, H)), _full((H, H))],
        out_specs=[_rows((R, H)), _rows((R, H)),
                   _rows((R, 1)), _rows((R, 1))],
        out_shape=[jax.ShapeDtypeStruct((N, H), jnp.float32),
                   jax.ShapeDtypeStruct((N, H), jnp.bfloat16),
                   jax.ShapeDtypeStruct((N, 1), jnp.float32),
                   jax.ShapeDtypeStruct((N, 1), jnp.float32)],
    )(degp3, x, W1, pmat)

    acc1 = _sc_aggregate(g1, src, dst, w)

    t2, g2 = pl.pallas_call(
        _tc_mid,
        grid=(G,),
        in_specs=[_accp_spec(), _rows((R, H)), _rows((R, 1)),
                  _rows((R, 1)), _full((1, H)), _full((H, H)),
                  _full((H, H))],
        out_specs=[_rows((R, H)), _rows((R, H))],
        out_shape=[jax.ShapeDtypeStruct((N, H), jnp.float32),
                   jax.ShapeDtypeStruct((N, H), jnp.bfloat16)],
    )(acc1, t1, dinv, dinv2, b1r, W2, pmat)

    acc2 = _sc_aggregate(g2, src, dst, w)

    h2, g3 = pl.pallas_call(
        _tc_pre_final,
        grid=(G,),
        in_specs=[_accp_spec(), _rows((R, H)), _rows((R, 1)),
                  _rows((R, 1)), _full((1, H)), _full((H, H))],
        out_specs=[_rows((R, H)), _rows((R, H))],
        out_shape=[jax.ShapeDtypeStruct((N, H), jnp.float32),
                   jax.ShapeDtypeStruct((N, H), jnp.bfloat16)],
    )(acc2, t2, dinv, dinv2, b2r, pmat)

    acc3 = _sc_aggregate(g3, src, dst, w)

    out = pl.pallas_call(
        _tc_final,
        grid=(G,),
        in_specs=[_accp_spec(), _rows((R, H)), _rows((R, 1)),
                  _rows((R, 1)), _full((1, D_OUT)), _full((H, D_OUT))],
        out_specs=_rows((R, D_OUT)),
        out_shape=jax.ShapeDtypeStruct((N, D_OUT), jnp.float32),
    )(acc3, h2, dinv, dinv2, b3r, W3)

    return out


# speedup vs baseline: 23.8338x; 1.1388x over previous
import functools

import jax
import jax.numpy as jnp
from jax import lax
from jax.experimental import pallas as pl
from jax.experimental.pallas import tpu as pltpu
from jax.experimental.pallas import tpu_sc as plsc

N = 10000
E = 320000
D_IN = 128
H = 64
D_OUT = 128

NC = 2
NS = 16
NW = NC * NS
CHUNK = 128
CPW = 80
EPW = CHUNK * CPW
E_PAD = NW * EPW
N_PAD = 10240
ROWS_PER_S = N_PAD // NS

_mesh = plsc.VectorSubcoreMesh(core_axis_name="c", subcore_axis_name="s")


@functools.partial(
    pl.kernel,
    out_type=jax.ShapeDtypeStruct((NC, N_PAD), jnp.float32),
    mesh=_mesh,
    scratch_types=[
        pltpu.VMEM_SHARED((N_PAD,), jnp.float32),
        pltpu.VMEM((CPW, CHUNK), jnp.int32),
        pltpu.VMEM((CPW, CHUNK), jnp.float32),
        pltpu.VMEM((ROWS_PER_S,), jnp.float32),
        pltpu.SemaphoreType.DMA,
    ],
    compiler_params=pltpu.CompilerParams(use_tc_tiling_on_sc=False),
)
def _sc_degree(dst_hbm, w_hbm, out_hbm, acc, dstb, wb, zbuf, sem):
    c = lax.axis_index("c")
    s = lax.axis_index("s")
    wid = s * NC + c

    def _z(i, _):
        zbuf[pl.ds(i * 16, 16)] = jnp.zeros((16,), jnp.float32)
        return 0
    lax.fori_loop(0, ROWS_PER_S // 16, _z, 0)
    pltpu.sync_copy(zbuf, acc.at[pl.ds(s * ROWS_PER_S, ROWS_PER_S)])

    pltpu.sync_copy(dst_hbm.at[pl.ds(wid * CPW, CPW)], dstb)
    pltpu.sync_copy(w_hbm.at[pl.ds(wid * CPW, CPW)], wb)
    plsc.subcore_barrier()

    def _chunk(ci, _):
        pltpu.async_copy(wb.at[ci], acc.at[dstb.at[ci]], sem, add=True)
        return 0
    lax.fori_loop(0, CPW, _chunk, 0)

    def _drain(ci, _):
        pltpu.make_async_copy(wb.at[ci], acc.at[dstb.at[ci]], sem).wait()
        return 0
    lax.fori_loop(0, CPW, _drain, 0)

    plsc.subcore_barrier()
    pltpu.sync_copy(acc.at[pl.ds(s * ROWS_PER_S, ROWS_PER_S)],
                    out_hbm.at[c, pl.ds(s * ROWS_PER_S, ROWS_PER_S)])


@functools.partial(
    pl.kernel,
    out_type=jax.ShapeDtypeStruct((NC, N_PAD, H), jnp.float32),
    mesh=_mesh,
    scratch_types=[
        pltpu.VMEM_SHARED((N_PAD, H), jnp.float32),
        pltpu.VMEM((CPW, CHUNK), jnp.int32),
        pltpu.VMEM((CPW, CHUNK), jnp.int32),
        pltpu.VMEM((CPW, CHUNK), jnp.float32),
        [pltpu.VMEM((CHUNK, H), jnp.bfloat16) for _ in range(4)],
        [pltpu.VMEM((CHUNK, H), jnp.float32) for _ in range(2)],
        [pltpu.SemaphoreType.DMA for _ in range(4)],
        [pltpu.SemaphoreType.DMA for _ in range(2)],
    ],
    compiler_params=pltpu.CompilerParams(use_tc_tiling_on_sc=False,
                                         needs_layout_passes=False),
)
def _sc_aggregate(g_hbm, src_hbm, dst_hbm, w_hbm, out_hbm,
                  acc, srcb, dstb, wb, gbufs, sbufs, gsems, ssems):
    c = lax.axis_index("c")
    s = lax.axis_index("s")
    wid = s * NC + c

    def _z(i, _):
        for j in range(H // 16):
            sbufs[0][i, pl.ds(j * 16, 16)] = jnp.zeros((16,), jnp.float32)
        return 0
    lax.fori_loop(0, CHUNK, _z, 0)
    for k in range(ROWS_PER_S // CHUNK):
        pltpu.sync_copy(sbufs[0],
                        acc.at[pl.ds(s * ROWS_PER_S + k * CHUNK, CHUNK)])

    pltpu.sync_copy(src_hbm.at[pl.ds(wid * CPW, CPW)], srcb)
    pltpu.sync_copy(dst_hbm.at[pl.ds(wid * CPW, CPW)], dstb)
    pltpu.sync_copy(w_hbm.at[pl.ds(wid * CPW, CPW)], wb)
    plsc.subcore_barrier()

    for k in range(4):
        pltpu.async_copy(g_hbm.at[srcb.at[k]], gbufs[k], gsems[k])

    def _quad(i, _):
        for k in range(4):
            ci = 4 * i + k
            gb, gsem = gbufs[k], gsems[k]
            sb, ssem = sbufs[k % 2], ssems[k % 2]
            pltpu.make_async_copy(g_hbm.at[srcb.at[ci]], gb, gsem).wait()

            def _drain_sb():
                pltpu.make_async_copy(sb, acc.at[dstb.at[ci]], ssem).wait()
            if k >= 2:
                _drain_sb()
            else:
                pl.when(i > 0)(_drain_sb)

            @functools.partial(plsc.parallel_loop, 0, CHUNK // 16, unroll=2)
            def _scale(gi):
                wv = wb[ci, pl.ds(gi * 16, 16)]
                for el in range(16):
                    e = gi * 16 + el
                    we = wv[el]
                    for h2 in range(2):
                        v = gb[e, pl.ds(32 * h2, 32)]
                        u = plsc.bitcast(v, jnp.uint32)
                        lo = plsc.bitcast(u << 16, jnp.float32)
                        hi = plsc.bitcast(u & jnp.uint32(0xFFFF0000),
                                          jnp.float32)
                        sb[e, pl.ds(32 * h2, 16)] = lo * we
                        sb[e, pl.ds(32 * h2 + 16, 16)] = hi * we

            pltpu.async_copy(sb, acc.at[dstb.at[ci]], ssem, add=True)

            @pl.when(ci + 4 < CPW)
            def _():
                pltpu.async_copy(g_hbm.at[srcb.at[ci + 4]], gb, gsem)
        return 0
    lax.fori_loop(0, CPW // 4, _quad, 0)

    for k in range(2):
        pltpu.make_async_copy(sbufs[k], acc.at[dstb.at[0]], ssems[k]).wait()

    plsc.subcore_barrier()
    pltpu.sync_copy(acc.at[pl.ds(s * ROWS_PER_S, ROWS_PER_S)],
                    out_hbm.at[c, pl.ds(s * ROWS_PER_S, ROWS_PER_S)])


R = 5000
G = N // R


def _tc_first(deg_ref, x_ref, w1_ref, p_ref, t1_ref, g1_ref,
              dinv_ref, dinv2_ref):
    deg = deg_ref[0] + deg_ref[1] + 1.0
    dinv = jnp.where(deg > 0,
                     lax.rsqrt(jnp.maximum(deg, 1e-12)), 0.0)
    t = jnp.dot(x_ref[...], w1_ref[...],
                preferred_element_type=jnp.float32)
    t1_ref[...] = t
    tp = jnp.dot(t, p_ref[...], preferred_element_type=jnp.float32)
    g1_ref[...] = (tp * dinv).astype(jnp.bfloat16)
    dinv_ref[...] = dinv
    dinv2_ref[...] = dinv * dinv


def _tc_mid(accp_ref, t_ref, dinv_ref, dinv2_ref, b_ref, w_ref, p_ref,
            tn_ref, gn_ref):
    a = (dinv_ref[...] * (accp_ref[0] + accp_ref[1])
         + dinv2_ref[...] * t_ref[...] + b_ref[...])
    h = jnp.maximum(a, 0.0)
    t = jnp.dot(h, w_ref[...], preferred_element_type=jnp.float32)
    tn_ref[...] = t
    tp = jnp.dot(t, p_ref[...], preferred_element_type=jnp.float32)
    gn_ref[...] = (tp * dinv_ref[...]).astype(jnp.bfloat16)


def _tc_pre_final(accp_ref, t_ref, dinv_ref, dinv2_ref, b_ref, p_ref,
                  h_ref, gn_ref):
    a = (dinv_ref[...] * (accp_ref[0] + accp_ref[1])
         + dinv2_ref[...] * t_ref[...] + b_ref[...])
    h = jnp.maximum(a, 0.0)
    h_ref[...] = h
    hp = jnp.dot(h, p_ref[...], preferred_element_type=jnp.float32)
    gn_ref[...] = (hp * dinv_ref[...]).astype(jnp.bfloat16)


def _tc_final(accp_ref, h_ref, dinv_ref, dinv2_ref, b_ref, w_ref, out_ref):
    a = (dinv_ref[...] * (accp_ref[0] + accp_ref[1])
         + dinv2_ref[...] * h_ref[...])
    out_ref[...] = (jnp.dot(a, w_ref[...], preferred_element_type=jnp.float32)
                    + b_ref[...])


def _rows(block_shape):
    return pl.BlockSpec(block_shape, lambda i: (i,) + (0,) * (len(block_shape) - 1))


def _accp_spec():
    return pl.BlockSpec((NC, R, H), lambda i: (0, i, 0))


def _full(shape):
    return pl.BlockSpec(shape, lambda i: (0,) * len(shape))


_PERM = [0] * H
for _h in (0, 1):
    for _k in range(16):
        _PERM[32 * _h + 2 * _k] = 32 * _h + _k
        _PERM[32 * _h + 2 * _k + 1] = 32 * _h + 16 + _k


def kernel(x, edge_index, edge_weight, W1, b1, W2, b2, W3, b3):
    pad = E_PAD - E
    src = jnp.concatenate([edge_index[0], jnp.zeros((pad,), jnp.int32)])
    dst = jnp.concatenate([edge_index[1], jnp.zeros((pad,), jnp.int32)])
    w = jnp.concatenate([edge_weight, jnp.zeros((pad,), jnp.float32)])
    src = src.reshape(NW * CPW, CHUNK)
    dst = dst.reshape(NW * CPW, CHUNK)
    w = w.reshape(NW * CPW, CHUNK)

    degp = _sc_degree(dst, w)
    degp3 = degp.reshape(NC, N_PAD, 1)

    b1r = b1.reshape(1, H)
    b2r = b2.reshape(1, H)
    b3r = b3.reshape(1, D_OUT)

    pmat = jnp.eye(H, dtype=jnp.float32)[jnp.array(_PERM, jnp.int32)].T

    t1, g1, dinv, dinv2 = pl.pallas_call(
        _tc_first,
        grid=(G,),
        in_specs=[pl.BlockSpec((NC, R, 1), lambda i: (0, i, 0)),
                  _rows((R, D_IN)), _full((D_IN, H)), _full((H, H))],
        out_specs=[_rows((R, H)), _rows((R, H)),
                   _rows((R, 1)), _rows((R, 1))],
        out_shape=[jax.ShapeDtypeStruct((N, H), jnp.float32),
                   jax.ShapeDtypeStruct((N, H), jnp.bfloat16),
                   jax.ShapeDtypeStruct((N, 1), jnp.float32),
                   jax.ShapeDtypeStruct((N, 1), jnp.float32)],
    )(degp3, x, W1, pmat)

    acc1 = _sc_aggregate(g1, src, dst, w)

    t2, g2 = pl.pallas_call(
        _tc_mid,
        grid=(G,),
        in_specs=[_accp_spec(), _rows((R, H)), _rows((R, 1)),
                  _rows((R, 1)), _full((1, H)), _full((H, H)),
                  _full((H, H))],
        out_specs=[_rows((R, H)), _rows((R, H))],
        out_shape=[jax.ShapeDtypeStruct((N, H), jnp.float32),
                   jax.ShapeDtypeStruct((N, H), jnp.bfloat16)],
    )(acc1, t1, dinv, dinv2, b1r, W2, pmat)

    acc2 = _sc_aggregate(g2, src, dst, w)

    h2, g3 = pl.pallas_call(
        _tc_pre_final,
        grid=(G,),
        in_specs=[_accp_spec(), _rows((R, H)), _rows((R, 1)),
                  _rows((R, 1)), _full((1, H)), _full((H, H))],
        out_specs=[_rows((R, H)), _rows((R, H))],
        out_shape=[jax.ShapeDtypeStruct((N, H), jnp.float32),
                   jax.ShapeDtypeStruct((N, H), jnp.bfloat16)],
    )(acc2, t2, dinv, dinv2, b2r, pmat)

    acc3 = _sc_aggregate(g3, src, dst, w)

    out = pl.pallas_call(
        _tc_final,
        grid=(G,),
        in_specs=[_accp_spec(), _rows((R, H)), _rows((R, 1)),
                  _rows((R, 1)), _full((1, D_OUT)), _full((H, D_OUT))],
        out_specs=_rows((R, D_OUT)),
        out_shape=jax.ShapeDtypeStruct((N, D_OUT), jnp.float32),
    )(acc3, h2, dinv, dinv2, b3r, W3)

    return out
